# baseline (speedup 1.0000x reference)
import functools

import jax
import jax.numpy as jnp
from jax import lax
from jax.experimental import pallas as pl
from jax.experimental.pallas import tpu as pltpu
from jax.experimental.pallas import tpu_sc as plsc

FEATURES = 64
B_TOK = 4096 * 50
LANE = 128
NC, NS = 2, 16
NW = NC * NS
ROWS_PER_W = B_TOK // (NW * LANE)
NJ = 10
NCHUNK = ROWS_PER_W // NJ
CHUNK = NJ * LANE


def _body(idx_hbm, table_hbm, out_hbm, idx_v, rows_v, sem):
    wid = lax.axis_index("s") * NC + lax.axis_index("c")
    base = wid * ROWS_PER_W * LANE

    def chunk(c, _):
        off = base + c * CHUNK
        pltpu.sync_copy(idx_hbm.at[pl.ds(off, CHUNK)], idx_v)
        handles = []
        for j in range(NJ):
            handles.append(pltpu.async_copy(
                table_hbm.at[idx_v.at[pl.ds(j * LANE, LANE)]],
                rows_v.at[pl.ds(j * LANE, LANE)],
                sem))
        for h in handles:
            h.wait()
        pltpu.sync_copy(rows_v, out_hbm.at[pl.ds(off, CHUNK)])
        return _

    lax.fori_loop(0, NCHUNK, chunk, 0)


@functools.partial(jax.jit, static_argnums=())
def _gather(idx2d, table):
    mesh = plsc.VectorSubcoreMesh(core_axis_name="c", subcore_axis_name="s")
    fn = functools.partial(
        pl.kernel,
        out_type=jax.ShapeDtypeStruct((B_TOK, FEATURES), jnp.float32),
        mesh=mesh,
        scratch_types=[
            pltpu.VMEM((CHUNK,), jnp.int32),
            pltpu.VMEM((CHUNK, FEATURES), jnp.float32),
            pltpu.SemaphoreType.DMA,
        ],
        compiler_params=pltpu.CompilerParams(use_tc_tiling_on_sc=False),
    )(_body)
    return fn(idx2d, table)


def kernel(inputs, embedding):
    idx2d = inputs.reshape(B_TOK).astype(jnp.int32)
    out = _gather(idx2d, embedding)
    return out.reshape(inputs.shape + (embedding.shape[-1],))


# speedup vs baseline: 1.0009x; 1.0009x over previous
import functools

import jax
import jax.numpy as jnp
from jax import lax
from jax.experimental import pallas as pl
from jax.experimental.pallas import tpu as pltpu
from jax.experimental.pallas import tpu_sc as plsc

FEATURES = 64
B_TOK = 4096 * 50
LANE = 128
NC, NS = 2, 16
NW = NC * NS
B_PER_W = B_TOK // NW
NJ = 5
CHUNK = NJ * LANE
NCHUNK = B_PER_W // CHUNK


def _body(idx_hbm, table_hbm, out_hbm, idx_v, rows0, rows1,
          sg0, sg1, so0, so1):
    wid = lax.axis_index("s") * NC + lax.axis_index("c")
    base = wid * B_PER_W
    pltpu.sync_copy(idx_hbm.at[pl.ds(base, B_PER_W)], idx_v)

    rows = (rows0, rows1)
    sg = (sg0, sg1)
    so = (so0, so1)
    gather_h = {}
    write_h = {}

    def fire_gathers(c):
        b = c % 2
        hs = []
        for j in range(NJ):
            k = c * NJ + j
            hs.append(pltpu.async_copy(
                table_hbm.at[idx_v.at[pl.ds(k * LANE, LANE)]],
                rows[b].at[pl.ds(j * LANE, LANE)],
                sg[b]))
        gather_h[c] = hs

    fire_gathers(0)
    for c in range(NCHUNK):
        b = c % 2
        if c + 1 < NCHUNK:
            if c >= 1:
                write_h[c - 1].wait()
            fire_gathers(c + 1)
        for h in gather_h[c]:
            h.wait()
        write_h[c] = pltpu.async_copy(
            rows[b], out_hbm.at[pl.ds(base + c * CHUNK, CHUNK)], so[b])
    write_h[NCHUNK - 2].wait()
    write_h[NCHUNK - 1].wait()


@jax.jit
def _gather(idx1d, table):
    mesh = plsc.VectorSubcoreMesh(core_axis_name="c", subcore_axis_name="s")
    fn = functools.partial(
        pl.kernel,
        out_type=jax.ShapeDtypeStruct((B_TOK, FEATURES), jnp.float32),
        mesh=mesh,
        scratch_types=[
            pltpu.VMEM((B_PER_W,), jnp.int32),
            pltpu.VMEM((CHUNK, FEATURES), jnp.float32),
            pltpu.VMEM((CHUNK, FEATURES), jnp.float32),
            pltpu.SemaphoreType.DMA,
            pltpu.SemaphoreType.DMA,
            pltpu.SemaphoreType.DMA,
            pltpu.SemaphoreType.DMA,
        ],
        compiler_params=pltpu.CompilerParams(use_tc_tiling_on_sc=False),
    )(_body)
    return fn(idx1d, table)


def kernel(inputs, embedding):
    idx1d = inputs.reshape(B_TOK).astype(jnp.int32)
    out = _gather(idx1d, embedding)
    return out.reshape(inputs.shape + (embedding.shape[-1],))
